# baseline (speedup 1.0000x reference)
import functools

import jax
import jax.numpy as jnp
from jax import lax
from jax.experimental import pallas as pl
from jax.experimental.pallas import tpu as pltpu
from jax.experimental.pallas import tpu_sc as plsc

_N = 10000
_E = 320000
_D = 128
_G = 64
_C = 10

_NPAD = 10240
_WORKERS = 32
_CW = 128
_CHUNKS = 79
_EPAD = _WORKERS * _CHUNKS * _CW
_CH0 = 98
_CH1 = 60
_CHH = 58
_NH0 = (58, 40)
_NH1 = (58, 2)
_CHMAX = max(_CH0, _CH1)
_RPS = _NPAD // 16

_BLK = 640
_GRID = _NPAD // _BLK


@functools.lru_cache(maxsize=None)
def _sc_kernels():
    mesh = plsc.VectorSubcoreMesh(core_axis_name="c", subcore_axis_name="s")

    @functools.partial(
        pl.kernel,
        mesh=mesh,
        out_type=jax.ShapeDtypeStruct((2, _NPAD, _D), jnp.float32),
        scratch_types=[
            pltpu.VMEM((_CHUNKS, _CW), jnp.int32),
            pltpu.VMEM((_CW, _D), jnp.float32),
            pltpu.VMEM_SHARED((_NPAD, _D), jnp.float32),
            pltpu.SemaphoreType.DMA,
        ],
    )
    def _deg_pass(dst_hbm, ones_hbm, zeros_hbm, out_hbm,
                  dst_v, ones_v, acc_sh, sem):
        c = lax.axis_index("c")
        s = lax.axis_index("s")
        wid = s * 2 + c
        pltpu.sync_copy(ones_hbm, ones_v)
        pltpu.sync_copy(dst_hbm.at[wid], dst_v)
        pltpu.sync_copy(zeros_hbm.at[pl.ds(s * _RPS, _RPS)],
                        acc_sh.at[pl.ds(s * _RPS, _RPS)])
        plsc.subcore_barrier()

        def body(j, carry):
            pltpu.sync_copy(ones_v, acc_sh.at[dst_v.at[j]], add=True)
            return carry

        lax.fori_loop(0, _CHUNKS, body, 0)
        plsc.subcore_barrier()
        pltpu.sync_copy(acc_sh.at[pl.ds(s * _RPS, _RPS)],
                        out_hbm.at[c, pl.ds(s * _RPS, _RPS)])

    @functools.partial(
        pl.kernel,
        mesh=mesh,
        out_type=jax.ShapeDtypeStruct((2, _NPAD, _D), jnp.float32),
        scratch_types=[
            pltpu.VMEM((_CHH, _CW), jnp.int32),
            pltpu.VMEM((_CHH, _CW), jnp.int32),
            pltpu.VMEM((2, _CW, _D), jnp.float32),
            pltpu.VMEM_SHARED((_NPAD, _D), jnp.float32),
            pltpu.SemaphoreType.DMA,
            pltpu.SemaphoreType.DMA,
        ],
    )
    def _edge_pass(u_hbm, src_hbm, dst_hbm, zeros_hbm, out_hbm,
                   src_v, dst_v, rows_v, acc_sh, sem0, sem1):
        c = lax.axis_index("c")
        s = lax.axis_index("s")
        wid = s * 2 + c
        pltpu.sync_copy(zeros_hbm.at[pl.ds(s * _RPS, _RPS)],
                        acc_sh.at[pl.ds(s * _RPS, _RPS)])
        plsc.subcore_barrier()

        def body(t, carry):
            j0 = 2 * t
            j1 = j0 + 1
            cp0 = pltpu.async_copy(u_hbm.at[src_v.at[j0]], rows_v.at[0], sem0)
            cp1 = pltpu.async_copy(u_hbm.at[src_v.at[j1]], rows_v.at[1], sem1)
            cp0.wait()
            pltpu.sync_copy(rows_v.at[0], acc_sh.at[dst_v.at[j0]], add=True)
            cp1.wait()
            pltpu.sync_copy(rows_v.at[1], acc_sh.at[dst_v.at[j1]], add=True)
            return carry

        for h in range(2):
            nh = jnp.where(c == 0, _NH0[h], _NH1[h])

            @pl.when(nh > 0)
            def _(h=h, nh=nh):
                pltpu.sync_copy(src_hbm.at[wid, h], src_v)
                pltpu.sync_copy(dst_hbm.at[wid, h], dst_v)
                lax.fori_loop(0, nh // 2, body, 0)

        plsc.subcore_barrier()
        pltpu.sync_copy(acc_sh.at[pl.ds(s * _RPS, _RPS)],
                        out_hbm.at[c, pl.ds(s * _RPS, _RPS)])

    return _deg_pass, _edge_pass


def _tc_first_body(deg_ref, x_ref, w_ref, dinv_ref, u_ref):
    deg = deg_ref[0, :, 0:1] + deg_ref[1, :, 0:1] + 1.0
    v = lax.rsqrt(deg)
    vb = jnp.broadcast_to(v, (_BLK, _D))
    dinv_ref[...] = vb
    u_ref[...] = vb * jnp.dot(x_ref[...], w_ref[...],
                              preferred_element_type=jnp.float32)


_tc_first = pl.pallas_call(
    _tc_first_body,
    grid=(_GRID,),
    in_specs=[
        pl.BlockSpec((2, _BLK, _D), lambda i: (0, i, 0)),
        pl.BlockSpec((_BLK, _D), lambda i: (i, 0)),
        pl.BlockSpec((_D, _D), lambda i: (0, 0)),
    ],
    out_specs=[
        pl.BlockSpec((_BLK, _D), lambda i: (i, 0)),
        pl.BlockSpec((_BLK, _D), lambda i: (i, 0)),
    ],
    out_shape=[
        jax.ShapeDtypeStruct((_NPAD, _D), jnp.float32),
        jax.ShapeDtypeStruct((_NPAD, _D), jnp.float32),
    ],
)


def _tc_mid_body(acc_ref, u_ref, dinv_ref, b_ref, w_ref, out_ref):
    vb = dinv_ref[...]
    cc = vb * (acc_ref[0] + acc_ref[1] + u_ref[...]) + b_ref[...]
    h = jnp.maximum(cc, 0.0)
    out_ref[...] = vb * jnp.dot(h, w_ref[...],
                                preferred_element_type=jnp.float32)


_tc_mid = pl.pallas_call(
    _tc_mid_body,
    grid=(_GRID,),
    in_specs=[
        pl.BlockSpec((2, _BLK, _D), lambda i: (0, i, 0)),
        pl.BlockSpec((_BLK, _D), lambda i: (i, 0)),
        pl.BlockSpec((_BLK, _D), lambda i: (i, 0)),
        pl.BlockSpec((1, _D), lambda i: (0, 0)),
        pl.BlockSpec((_D, _D), lambda i: (0, 0)),
    ],
    out_specs=pl.BlockSpec((_BLK, _D), lambda i: (i, 0)),
    out_shape=jax.ShapeDtypeStruct((_NPAD, _D), jnp.float32),
)


def _tc_final_body(acc_ref, u_ref, dinv_ref, b4_ref, batch_ref,
                   wl1_ref, bl1_ref, gamma_ref, beta_ref, wl2_ref, bl2_ref,
                   out_ref, g_scr):
    i = pl.program_id(0)
    c4 = dinv_ref[...] * (acc_ref[0] + acc_ref[1] + u_ref[...]) + b4_ref[...]
    bb = batch_ref[0]
    gids = lax.broadcasted_iota(jnp.int32, (_G, _BLK), 0)
    onehot = (bb == gids).astype(jnp.float32)
    part = jnp.dot(onehot, c4, preferred_element_type=jnp.float32,
                   precision=lax.Precision.HIGHEST)

    @pl.when(i == 0)
    def _():
        g_scr[...] = part

    @pl.when(i > 0)
    def _():
        g_scr[...] = g_scr[...] + part

    @pl.when(i == pl.num_programs(0) - 1)
    def _():
        g = g_scr[...]
        z = jnp.dot(g, wl1_ref[...], preferred_element_type=jnp.float32)
        z = z + bl1_ref[...]
        mean = jnp.mean(z, axis=0, keepdims=True)
        var = jnp.mean((z - mean) * (z - mean), axis=0, keepdims=True)
        z = (z - mean) * lax.rsqrt(var + 1e-5) * gamma_ref[...] + beta_ref[...]
        z = jnp.maximum(z, 0.0)
        o = jnp.dot(z, wl2_ref[...], preferred_element_type=jnp.float32)
        o = o + bl2_ref[...]
        colmask = lax.broadcasted_iota(jnp.int32, (_G, _D), 1) < _C
        om = jnp.where(colmask, o, -jnp.inf)
        m = jnp.max(om, axis=1, keepdims=True)
        e = jnp.where(colmask, jnp.exp(o - m), 0.0)
        lse = jnp.log(jnp.sum(e, axis=1, keepdims=True))
        out_ref[...] = o - m - lse


_tc_final = pl.pallas_call(
    _tc_final_body,
    grid=(_GRID,),
    in_specs=[
        pl.BlockSpec((2, _BLK, _D), lambda i: (0, i, 0)),
        pl.BlockSpec((_BLK, _D), lambda i: (i, 0)),
        pl.BlockSpec((_BLK, _D), lambda i: (i, 0)),
        pl.BlockSpec((1, _D), lambda i: (0, 0)),
        pl.BlockSpec((1, 1, _BLK), lambda i: (i, 0, 0)),
        pl.BlockSpec((_D, _D), lambda i: (0, 0)),
        pl.BlockSpec((1, _D), lambda i: (0, 0)),
        pl.BlockSpec((1, _D), lambda i: (0, 0)),
        pl.BlockSpec((1, _D), lambda i: (0, 0)),
        pl.BlockSpec((_D, _D), lambda i: (0, 0)),
        pl.BlockSpec((1, _D), lambda i: (0, 0)),
    ],
    out_specs=pl.BlockSpec((_G, _D), lambda i: (0, 0)),
    out_shape=jax.ShapeDtypeStruct((_G, _D), jnp.float32),
    scratch_shapes=[pltpu.VMEM((_G, _D), jnp.float32)],
)


def kernel(x, edge_index, batch, W1, b1, W2, b2, W3, b3, W4, b4,
           Wl1, bl1, gamma, beta, Wl2, bl2):
    src = edge_index[0]
    dst = edge_index[1]
    idx_pad = jnp.full((_EPAD - _E,), _N, jnp.int32)
    srcp = jnp.concatenate([src, idx_pad]).reshape(_WORKERS, _CHUNKS, _CW)
    dstp = jnp.concatenate([dst, idx_pad]).reshape(_WORKERS, _CHUNKS, _CW)

    def _skew(e):
        ep = jnp.concatenate([e, idx_pad])
        n0 = 16 * _CH0 * _CW
        e0 = ep[:n0].reshape(16, _CH0, _CW)
        e0 = jnp.pad(e0, ((0, 0), (0, 2 * _CHH - _CH0), (0, 0)),
                     constant_values=_N)
        e1 = ep[n0:].reshape(16, _CH1, _CW)
        e1 = jnp.pad(e1, ((0, 0), (0, 2 * _CHH - _CH1), (0, 0)),
                     constant_values=_N)
        return jnp.stack([e0, e1], axis=1).reshape(_WORKERS, 2, _CHH, _CW)

    srcq = _skew(src)
    dstq = _skew(dst)

    xp = jnp.pad(x, ((0, _NPAD - _N), (0, 0)))
    batchp = jnp.pad(batch, (0, _NPAD - _N),
                     constant_values=_G).reshape(_GRID, 1, _BLK)
    zeros = jnp.zeros((_NPAD, _D), jnp.float32)
    ones_rows = jnp.ones((_CW, _D), jnp.float32)

    b1r = b1.reshape(1, _D)
    b2r = b2.reshape(1, _D)
    b3r = b3.reshape(1, _D)
    b4r = b4.reshape(1, _D)
    bl1r = bl1.reshape(1, _D)
    gammar = gamma.reshape(1, _D)
    betar = beta.reshape(1, _D)
    Wl2p = jnp.pad(Wl2, ((0, 0), (0, _D - _C)))
    bl2p = jnp.pad(bl2, (0, _D - _C)).reshape(1, _D)

    _deg_pass, _edge_pass = _sc_kernels()
    deg2 = _deg_pass(dstp, ones_rows, zeros)
    dinvb, u = _tc_first(deg2, xp, W1)

    acc = _edge_pass(u, srcq, dstq, zeros)
    u = _tc_mid(acc, u, dinvb, b1r, W2)
    acc = _edge_pass(u, srcq, dstq, zeros)
    u = _tc_mid(acc, u, dinvb, b2r, W3)
    acc = _edge_pass(u, srcq, dstq, zeros)
    u = _tc_mid(acc, u, dinvb, b3r, W4)
    acc = _edge_pass(u, srcq, dstq, zeros)

    out = _tc_final(acc, u, dinvb, b4r, batchp,
                    Wl1, bl1r, gammar, betar, Wl2p, bl2p)
    return out[:, :_C]


# speedup vs baseline: 1.1307x; 1.1307x over previous
import functools

import jax
import jax.numpy as jnp
from jax import lax
from jax.experimental import pallas as pl
from jax.experimental.pallas import tpu as pltpu
from jax.experimental.pallas import tpu_sc as plsc

_N = 10000
_E = 320000
_D = 128
_G = 64
_C = 10

_NPAD = 10240
_WORKERS = 32
_CW = 128
_CHUNKS = 79
_EPAD = _WORKERS * _CHUNKS * _CW
_CH0 = 114
_CH1 = 44
_CHH = 58
_NH0 = (58, 56)
_NH1 = (44, 0)
_CHMAX = max(_CH0, _CH1)
_RPS = _NPAD // 16

_BLK = 640
_GRID = _NPAD // _BLK


@functools.lru_cache(maxsize=None)
def _sc_kernels():
    mesh = plsc.VectorSubcoreMesh(core_axis_name="c", subcore_axis_name="s")

    @functools.partial(
        pl.kernel,
        mesh=mesh,
        out_type=jax.ShapeDtypeStruct((2, _NPAD, _D), jnp.float32),
        scratch_types=[
            pltpu.VMEM((_CHUNKS, _CW), jnp.int32),
            pltpu.VMEM((_CW, _D), jnp.float32),
            pltpu.VMEM_SHARED((_NPAD, _D), jnp.float32),
            pltpu.SemaphoreType.DMA,
        ],
    )
    def _deg_pass(dst_hbm, ones_hbm, zeros_hbm, out_hbm,
                  dst_v, ones_v, acc_sh, sem):
        c = lax.axis_index("c")
        s = lax.axis_index("s")
        wid = s * 2 + c
        pltpu.sync_copy(ones_hbm, ones_v)
        pltpu.sync_copy(dst_hbm.at[wid], dst_v)
        pltpu.sync_copy(zeros_hbm.at[pl.ds(s * _RPS, _RPS)],
                        acc_sh.at[pl.ds(s * _RPS, _RPS)])
        plsc.subcore_barrier()

        def body(j, carry):
            pltpu.sync_copy(ones_v, acc_sh.at[dst_v.at[j]], add=True)
            return carry

        lax.fori_loop(0, _CHUNKS, body, 0)
        plsc.subcore_barrier()
        pltpu.sync_copy(acc_sh.at[pl.ds(s * _RPS, _RPS)],
                        out_hbm.at[c, pl.ds(s * _RPS, _RPS)])

    @functools.partial(
        pl.kernel,
        mesh=mesh,
        out_type=jax.ShapeDtypeStruct((2, _NPAD, _D), jnp.float32),
        scratch_types=[
            pltpu.VMEM((_CHH, _CW), jnp.int32),
            pltpu.VMEM((_CHH, _CW), jnp.int32),
            pltpu.VMEM((2, _CW, _D), jnp.float32),
            pltpu.VMEM_SHARED((_NPAD, _D), jnp.float32),
            pltpu.SemaphoreType.DMA,
            pltpu.SemaphoreType.DMA,
        ],
    )
    def _edge_pass(u_hbm, src_hbm, dst_hbm, zeros_hbm, out_hbm,
                   src_v, dst_v, rows_v, acc_sh, sem0, sem1):
        c = lax.axis_index("c")
        s = lax.axis_index("s")
        wid = s * 2 + c
        pltpu.sync_copy(zeros_hbm.at[pl.ds(s * _RPS, _RPS)],
                        acc_sh.at[pl.ds(s * _RPS, _RPS)])
        plsc.subcore_barrier()

        def body(t, carry):
            j0 = 2 * t
            j1 = j0 + 1
            cp0 = pltpu.async_copy(u_hbm.at[src_v.at[j0]], rows_v.at[0], sem0)
            cp1 = pltpu.async_copy(u_hbm.at[src_v.at[j1]], rows_v.at[1], sem1)
            cp0.wait()
            pltpu.sync_copy(rows_v.at[0], acc_sh.at[dst_v.at[j0]], add=True)
            cp1.wait()
            pltpu.sync_copy(rows_v.at[1], acc_sh.at[dst_v.at[j1]], add=True)
            return carry

        for h in range(2):
            nh = jnp.where(c == 0, _NH0[h], _NH1[h])

            @pl.when(nh > 0)
            def _(h=h, nh=nh):
                pltpu.sync_copy(src_hbm.at[wid, h], src_v)
                pltpu.sync_copy(dst_hbm.at[wid, h], dst_v)
                lax.fori_loop(0, nh // 2, body, 0)

        plsc.subcore_barrier()
        pltpu.sync_copy(acc_sh.at[pl.ds(s * _RPS, _RPS)],
                        out_hbm.at[c, pl.ds(s * _RPS, _RPS)])

    return _deg_pass, _edge_pass


def _tc_first_body(deg_ref, x_ref, w_ref, dinv_ref, u_ref):
    deg = deg_ref[0, :, 0:1] + deg_ref[1, :, 0:1] + 1.0
    v = lax.rsqrt(deg)
    vb = jnp.broadcast_to(v, (_BLK, _D))
    dinv_ref[...] = vb
    u_ref[...] = vb * jnp.dot(x_ref[...], w_ref[...],
                              preferred_element_type=jnp.float32)


_tc_first = pl.pallas_call(
    _tc_first_body,
    grid=(_GRID,),
    in_specs=[
        pl.BlockSpec((2, _BLK, _D), lambda i: (0, i, 0)),
        pl.BlockSpec((_BLK, _D), lambda i: (i, 0)),
        pl.BlockSpec((_D, _D), lambda i: (0, 0)),
    ],
    out_specs=[
        pl.BlockSpec((_BLK, _D), lambda i: (i, 0)),
        pl.BlockSpec((_BLK, _D), lambda i: (i, 0)),
    ],
    out_shape=[
        jax.ShapeDtypeStruct((_NPAD, _D), jnp.float32),
        jax.ShapeDtypeStruct((_NPAD, _D), jnp.float32),
    ],
)


def _tc_mid_body(acc_ref, u_ref, dinv_ref, b_ref, w_ref, out_ref):
    vb = dinv_ref[...]
    cc = vb * (acc_ref[0] + acc_ref[1] + u_ref[...]) + b_ref[...]
    h = jnp.maximum(cc, 0.0)
    out_ref[...] = vb * jnp.dot(h, w_ref[...],
                                preferred_element_type=jnp.float32)


_tc_mid = pl.pallas_call(
    _tc_mid_body,
    grid=(_GRID,),
    in_specs=[
        pl.BlockSpec((2, _BLK, _D), lambda i: (0, i, 0)),
        pl.BlockSpec((_BLK, _D), lambda i: (i, 0)),
        pl.BlockSpec((_BLK, _D), lambda i: (i, 0)),
        pl.BlockSpec((1, _D), lambda i: (0, 0)),
        pl.BlockSpec((_D, _D), lambda i: (0, 0)),
    ],
    out_specs=pl.BlockSpec((_BLK, _D), lambda i: (i, 0)),
    out_shape=jax.ShapeDtypeStruct((_NPAD, _D), jnp.float32),
)


def _tc_final_body(acc_ref, u_ref, dinv_ref, b4_ref, batch_ref,
                   wl1_ref, bl1_ref, gamma_ref, beta_ref, wl2_ref, bl2_ref,
                   out_ref, g_scr):
    i = pl.program_id(0)
    c4 = dinv_ref[...] * (acc_ref[0] + acc_ref[1] + u_ref[...]) + b4_ref[...]
    bb = batch_ref[0]
    gids = lax.broadcasted_iota(jnp.int32, (_G, _BLK), 0)
    onehot = (bb == gids).astype(jnp.float32)
    part = jnp.dot(onehot, c4, preferred_element_type=jnp.float32,
                   precision=lax.Precision.HIGHEST)

    @pl.when(i == 0)
    def _():
        g_scr[...] = part

    @pl.when(i > 0)
    def _():
        g_scr[...] = g_scr[...] + part

    @pl.when(i == pl.num_programs(0) - 1)
    def _():
        g = g_scr[...]
        z = jnp.dot(g, wl1_ref[...], preferred_element_type=jnp.float32)
        z = z + bl1_ref[...]
        mean = jnp.mean(z, axis=0, keepdims=True)
        var = jnp.mean((z - mean) * (z - mean), axis=0, keepdims=True)
        z = (z - mean) * lax.rsqrt(var + 1e-5) * gamma_ref[...] + beta_ref[...]
        z = jnp.maximum(z, 0.0)
        o = jnp.dot(z, wl2_ref[...], preferred_element_type=jnp.float32)
        o = o + bl2_ref[...]
        colmask = lax.broadcasted_iota(jnp.int32, (_G, _D), 1) < _C
        om = jnp.where(colmask, o, -jnp.inf)
        m = jnp.max(om, axis=1, keepdims=True)
        e = jnp.where(colmask, jnp.exp(o - m), 0.0)
        lse = jnp.log(jnp.sum(e, axis=1, keepdims=True))
        out_ref[...] = o - m - lse


_tc_final = pl.pallas_call(
    _tc_final_body,
    grid=(_GRID,),
    in_specs=[
        pl.BlockSpec((2, _BLK, _D), lambda i: (0, i, 0)),
        pl.BlockSpec((_BLK, _D), lambda i: (i, 0)),
        pl.BlockSpec((_BLK, _D), lambda i: (i, 0)),
        pl.BlockSpec((1, _D), lambda i: (0, 0)),
        pl.BlockSpec((1, 1, _BLK), lambda i: (i, 0, 0)),
        pl.BlockSpec((_D, _D), lambda i: (0, 0)),
        pl.BlockSpec((1, _D), lambda i: (0, 0)),
        pl.BlockSpec((1, _D), lambda i: (0, 0)),
        pl.BlockSpec((1, _D), lambda i: (0, 0)),
        pl.BlockSpec((_D, _D), lambda i: (0, 0)),
        pl.BlockSpec((1, _D), lambda i: (0, 0)),
    ],
    out_specs=pl.BlockSpec((_G, _D), lambda i: (0, 0)),
    out_shape=jax.ShapeDtypeStruct((_G, _D), jnp.float32),
    scratch_shapes=[pltpu.VMEM((_G, _D), jnp.float32)],
)


def kernel(x, edge_index, batch, W1, b1, W2, b2, W3, b3, W4, b4,
           Wl1, bl1, gamma, beta, Wl2, bl2):
    src = edge_index[0]
    dst = edge_index[1]
    idx_pad = jnp.full((_EPAD - _E,), _N, jnp.int32)
    srcp = jnp.concatenate([src, idx_pad]).reshape(_WORKERS, _CHUNKS, _CW)
    dstp = jnp.concatenate([dst, idx_pad]).reshape(_WORKERS, _CHUNKS, _CW)

    def _skew(e):
        ep = jnp.concatenate([e, idx_pad])
        n0 = 16 * _CH0 * _CW
        e0 = ep[:n0].reshape(16, _CH0, _CW)
        e0 = jnp.pad(e0, ((0, 0), (0, 2 * _CHH - _CH0), (0, 0)),
                     constant_values=_N)
        e1 = ep[n0:].reshape(16, _CH1, _CW)
        e1 = jnp.pad(e1, ((0, 0), (0, 2 * _CHH - _CH1), (0, 0)),
                     constant_values=_N)
        return jnp.stack([e0, e1], axis=1).reshape(_WORKERS, 2, _CHH, _CW)

    srcq = _skew(src)
    dstq = _skew(dst)

    xp = jnp.pad(x, ((0, _NPAD - _N), (0, 0)))
    batchp = jnp.pad(batch, (0, _NPAD - _N),
                     constant_values=_G).reshape(_GRID, 1, _BLK)
    zeros = jnp.zeros((_NPAD, _D), jnp.float32)
    ones_rows = jnp.ones((_CW, _D), jnp.float32)

    b1r = b1.reshape(1, _D)
    b2r = b2.reshape(1, _D)
    b3r = b3.reshape(1, _D)
    b4r = b4.reshape(1, _D)
    bl1r = bl1.reshape(1, _D)
    gammar = gamma.reshape(1, _D)
    betar = beta.reshape(1, _D)
    Wl2p = jnp.pad(Wl2, ((0, 0), (0, _D - _C)))
    bl2p = jnp.pad(bl2, (0, _D - _C)).reshape(1, _D)

    _deg_pass, _edge_pass = _sc_kernels()
    deg2 = _deg_pass(dstp, ones_rows, zeros)
    dinvb, u = _tc_first(deg2, xp, W1)

    acc = _edge_pass(u, srcq, dstq, zeros)
    u = _tc_mid(acc, u, dinvb, b1r, W2)
    acc = _edge_pass(u, srcq, dstq, zeros)
    u = _tc_mid(acc, u, dinvb, b2r, W3)
    acc = _edge_pass(u, srcq, dstq, zeros)
    u = _tc_mid(acc, u, dinvb, b3r, W4)
    acc = _edge_pass(u, srcq, dstq, zeros)

    out = _tc_final(acc, u, dinvb, b4r, batchp,
                    Wl1, bl1r, gammar, betar, Wl2p, bl2p)
    return out[:, :_C]


# speedup vs baseline: 1.1821x; 1.0454x over previous
import functools

import jax
import jax.numpy as jnp
from jax import lax
from jax.experimental import pallas as pl
from jax.experimental.pallas import tpu as pltpu
from jax.experimental.pallas import tpu_sc as plsc

_N = 10000
_E = 320000
_D = 128
_G = 64
_C = 10

_NPAD = 10240
_WORKERS = 32
_CW = 128
_CHUNKS = 79
_EPAD = _WORKERS * _CHUNKS * _CW
_CH0 = 106
_CH1 = 52
_CHH = 54
_NH0 = (54, 52)
_NH1 = (52, 0)
_CHMAX = max(_CH0, _CH1)
_RPS = _NPAD // 16

_BLK = 640
_GRID = _NPAD // _BLK


@functools.lru_cache(maxsize=None)
def _sc_kernels():
    mesh = plsc.VectorSubcoreMesh(core_axis_name="c", subcore_axis_name="s")

    @functools.partial(
        pl.kernel,
        mesh=mesh,
        out_type=jax.ShapeDtypeStruct((2, _NPAD, _D), jnp.float32),
        scratch_types=[
            pltpu.VMEM((_CHUNKS, _CW), jnp.int32),
            pltpu.VMEM((_CW, _D), jnp.float32),
            pltpu.VMEM_SHARED((_NPAD, _D), jnp.float32),
            pltpu.SemaphoreType.DMA,
        ],
    )
    def _deg_pass(dst_hbm, ones_hbm, zeros_hbm, out_hbm,
                  dst_v, ones_v, acc_sh, sem):
        c = lax.axis_index("c")
        s = lax.axis_index("s")
        wid = s * 2 + c
        pltpu.sync_copy(ones_hbm, ones_v)
        pltpu.sync_copy(dst_hbm.at[wid], dst_v)
        pltpu.sync_copy(zeros_hbm.at[pl.ds(s * _RPS, _RPS)],
                        acc_sh.at[pl.ds(s * _RPS, _RPS)])
        plsc.subcore_barrier()

        def body(j, carry):
            pltpu.sync_copy(ones_v, acc_sh.at[dst_v.at[j]], add=True)
            return carry

        lax.fori_loop(0, _CHUNKS, body, 0)
        plsc.subcore_barrier()
        pltpu.sync_copy(acc_sh.at[pl.ds(s * _RPS, _RPS)],
                        out_hbm.at[c, pl.ds(s * _RPS, _RPS)])

    @functools.partial(
        pl.kernel,
        mesh=mesh,
        out_type=jax.ShapeDtypeStruct((2, _NPAD, _D), jnp.float32),
        scratch_types=[
            pltpu.VMEM((_CHH, _CW), jnp.int32),
            pltpu.VMEM((_CHH, _CW), jnp.int32),
            pltpu.VMEM((2, _CW, _D), jnp.float32),
            pltpu.VMEM_SHARED((_NPAD, _D), jnp.float32),
            pltpu.SemaphoreType.DMA,
            pltpu.SemaphoreType.DMA,
        ],
    )
    def _edge_pass(u_hbm, src_hbm, dst_hbm, zeros_hbm, out_hbm,
                   src_v, dst_v, rows_v, acc_sh, sem0, sem1):
        c = lax.axis_index("c")
        s = lax.axis_index("s")
        wid = s * 2 + c
        pltpu.sync_copy(zeros_hbm.at[pl.ds(s * _RPS, _RPS)],
                        acc_sh.at[pl.ds(s * _RPS, _RPS)])
        plsc.subcore_barrier()

        def body(t, carry):
            j0 = 2 * t
            j1 = j0 + 1
            cp0 = pltpu.async_copy(u_hbm.at[src_v.at[j0]], rows_v.at[0], sem0)
            cp1 = pltpu.async_copy(u_hbm.at[src_v.at[j1]], rows_v.at[1], sem1)
            cp0.wait()
            pltpu.sync_copy(rows_v.at[0], acc_sh.at[dst_v.at[j0]], add=True)
            cp1.wait()
            pltpu.sync_copy(rows_v.at[1], acc_sh.at[dst_v.at[j1]], add=True)
            return carry

        for h in range(2):
            nh = jnp.where(c == 0, _NH0[h], _NH1[h])

            @pl.when(nh > 0)
            def _(h=h, nh=nh):
                pltpu.sync_copy(src_hbm.at[wid, h], src_v)
                pltpu.sync_copy(dst_hbm.at[wid, h], dst_v)
                lax.fori_loop(0, nh // 2, body, 0)

        plsc.subcore_barrier()
        pltpu.sync_copy(acc_sh.at[pl.ds(s * _RPS, _RPS)],
                        out_hbm.at[c, pl.ds(s * _RPS, _RPS)])

    return _deg_pass, _edge_pass


def _tc_first_body(deg_ref, x_ref, w_ref, dinv_ref, u_ref):
    deg = deg_ref[0, :, 0:1] + deg_ref[1, :, 0:1] + 1.0
    v = lax.rsqrt(deg)
    vb = jnp.broadcast_to(v, (_BLK, _D))
    dinv_ref[...] = vb
    u_ref[...] = vb * jnp.dot(x_ref[...], w_ref[...],
                              preferred_element_type=jnp.float32)


_tc_first = pl.pallas_call(
    _tc_first_body,
    grid=(_GRID,),
    in_specs=[
        pl.BlockSpec((2, _BLK, _D), lambda i: (0, i, 0)),
        pl.BlockSpec((_BLK, _D), lambda i: (i, 0)),
        pl.BlockSpec((_D, _D), lambda i: (0, 0)),
    ],
    out_specs=[
        pl.BlockSpec((_BLK, _D), lambda i: (i, 0)),
        pl.BlockSpec((_BLK, _D), lambda i: (i, 0)),
    ],
    out_shape=[
        jax.ShapeDtypeStruct((_NPAD, _D), jnp.float32),
        jax.ShapeDtypeStruct((_NPAD, _D), jnp.float32),
    ],
)


def _tc_mid_body(acc_ref, u_ref, dinv_ref, b_ref, w_ref, out_ref):
    vb = dinv_ref[...]
    cc = vb * (acc_ref[0] + acc_ref[1] + u_ref[...]) + b_ref[...]
    h = jnp.maximum(cc, 0.0)
    out_ref[...] = vb * jnp.dot(h, w_ref[...],
                                preferred_element_type=jnp.float32)


_tc_mid = pl.pallas_call(
    _tc_mid_body,
    grid=(_GRID,),
    in_specs=[
        pl.BlockSpec((2, _BLK, _D), lambda i: (0, i, 0)),
        pl.BlockSpec((_BLK, _D), lambda i: (i, 0)),
        pl.BlockSpec((_BLK, _D), lambda i: (i, 0)),
        pl.BlockSpec((1, _D), lambda i: (0, 0)),
        pl.BlockSpec((_D, _D), lambda i: (0, 0)),
    ],
    out_specs=pl.BlockSpec((_BLK, _D), lambda i: (i, 0)),
    out_shape=jax.ShapeDtypeStruct((_NPAD, _D), jnp.float32),
)


def _tc_final_body(acc_ref, u_ref, dinv_ref, b4_ref, batch_ref,
                   wl1_ref, bl1_ref, gamma_ref, beta_ref, wl2_ref, bl2_ref,
                   out_ref, g_scr):
    i = pl.program_id(0)
    c4 = dinv_ref[...] * (acc_ref[0] + acc_ref[1] + u_ref[...]) + b4_ref[...]
    bb = batch_ref[0]
    gids = lax.broadcasted_iota(jnp.int32, (_G, _BLK), 0)
    onehot = (bb == gids).astype(jnp.float32)
    part = jnp.dot(onehot, c4, preferred_element_type=jnp.float32,
                   precision=lax.Precision.HIGHEST)

    @pl.when(i == 0)
    def _():
        g_scr[...] = part

    @pl.when(i > 0)
    def _():
        g_scr[...] = g_scr[...] + part

    @pl.when(i == pl.num_programs(0) - 1)
    def _():
        g = g_scr[...]
        z = jnp.dot(g, wl1_ref[...], preferred_element_type=jnp.float32)
        z = z + bl1_ref[...]
        mean = jnp.mean(z, axis=0, keepdims=True)
        var = jnp.mean((z - mean) * (z - mean), axis=0, keepdims=True)
        z = (z - mean) * lax.rsqrt(var + 1e-5) * gamma_ref[...] + beta_ref[...]
        z = jnp.maximum(z, 0.0)
        o = jnp.dot(z, wl2_ref[...], preferred_element_type=jnp.float32)
        o = o + bl2_ref[...]
        colmask = lax.broadcasted_iota(jnp.int32, (_G, _D), 1) < _C
        om = jnp.where(colmask, o, -jnp.inf)
        m = jnp.max(om, axis=1, keepdims=True)
        e = jnp.where(colmask, jnp.exp(o - m), 0.0)
        lse = jnp.log(jnp.sum(e, axis=1, keepdims=True))
        out_ref[...] = o - m - lse


_tc_final = pl.pallas_call(
    _tc_final_body,
    grid=(_GRID,),
    in_specs=[
        pl.BlockSpec((2, _BLK, _D), lambda i: (0, i, 0)),
        pl.BlockSpec((_BLK, _D), lambda i: (i, 0)),
        pl.BlockSpec((_BLK, _D), lambda i: (i, 0)),
        pl.BlockSpec((1, _D), lambda i: (0, 0)),
        pl.BlockSpec((1, 1, _BLK), lambda i: (i, 0, 0)),
        pl.BlockSpec((_D, _D), lambda i: (0, 0)),
        pl.BlockSpec((1, _D), lambda i: (0, 0)),
        pl.BlockSpec((1, _D), lambda i: (0, 0)),
        pl.BlockSpec((1, _D), lambda i: (0, 0)),
        pl.BlockSpec((_D, _D), lambda i: (0, 0)),
        pl.BlockSpec((1, _D), lambda i: (0, 0)),
    ],
    out_specs=pl.BlockSpec((_G, _D), lambda i: (0, 0)),
    out_shape=jax.ShapeDtypeStruct((_G, _D), jnp.float32),
    scratch_shapes=[pltpu.VMEM((_G, _D), jnp.float32)],
)


def kernel(x, edge_index, batch, W1, b1, W2, b2, W3, b3, W4, b4,
           Wl1, bl1, gamma, beta, Wl2, bl2):
    src = edge_index[0]
    dst = edge_index[1]
    idx_pad = jnp.full((_EPAD - _E,), _N, jnp.int32)
    srcp = jnp.concatenate([src, idx_pad]).reshape(_WORKERS, _CHUNKS, _CW)
    dstp = jnp.concatenate([dst, idx_pad]).reshape(_WORKERS, _CHUNKS, _CW)

    def _skew(e):
        ep = jnp.concatenate([e, idx_pad])
        n0 = 16 * _CH0 * _CW
        e0 = ep[:n0].reshape(16, _CH0, _CW)
        e0 = jnp.pad(e0, ((0, 0), (0, 2 * _CHH - _CH0), (0, 0)),
                     constant_values=_N)
        e1 = ep[n0:].reshape(16, _CH1, _CW)
        e1 = jnp.pad(e1, ((0, 0), (0, 2 * _CHH - _CH1), (0, 0)),
                     constant_values=_N)
        return jnp.stack([e0, e1], axis=1).reshape(_WORKERS, 2, _CHH, _CW)

    srcq = _skew(src)
    dstq = _skew(dst)

    xp = jnp.pad(x, ((0, _NPAD - _N), (0, 0)))
    batchp = jnp.pad(batch, (0, _NPAD - _N),
                     constant_values=_G).reshape(_GRID, 1, _BLK)
    zeros = jnp.zeros((_NPAD, _D), jnp.float32)
    ones_rows = jnp.ones((_CW, _D), jnp.float32)

    b1r = b1.reshape(1, _D)
    b2r = b2.reshape(1, _D)
    b3r = b3.reshape(1, _D)
    b4r = b4.reshape(1, _D)
    bl1r = bl1.reshape(1, _D)
    gammar = gamma.reshape(1, _D)
    betar = beta.reshape(1, _D)
    Wl2p = jnp.pad(Wl2, ((0, 0), (0, _D - _C)))
    bl2p = jnp.pad(bl2, (0, _D - _C)).reshape(1, _D)

    _deg_pass, _edge_pass = _sc_kernels()
    deg2 = _deg_pass(dstp, ones_rows, zeros)
    dinvb, u = _tc_first(deg2, xp, W1)

    acc = _edge_pass(u, srcq, dstq, zeros)
    u = _tc_mid(acc, u, dinvb, b1r, W2)
    acc = _edge_pass(u, srcq, dstq, zeros)
    u = _tc_mid(acc, u, dinvb, b2r, W3)
    acc = _edge_pass(u, srcq, dstq, zeros)
    u = _tc_mid(acc, u, dinvb, b3r, W4)
    acc = _edge_pass(u, srcq, dstq, zeros)

    out = _tc_final(acc, u, dinvb, b4r, batchp,
                    Wl1, bl1r, gammar, betar, Wl2p, bl2p)
    return out[:, :_C]
